# baseline (speedup 1.0000x reference)
import functools
import jax
import jax.numpy as jnp
from jax import lax
from jax.experimental import pallas as pl
from jax.experimental.pallas import tpu as pltpu

B = 16384
D = 64
CHT = 256


def _tc_body(uidx_s, iidx_s, uw_hbm, iw_hbm, out_v, ubuf, ibuf, sems):
    k = pl.program_id(0)
    for j in range(CHT):
        r_u = uidx_s[k * CHT + j]
        r_i = iidx_s[k * CHT + j]
        pltpu.make_async_copy(
            uw_hbm.at[pl.ds(r_u, 1)], ubuf.at[pl.ds(j, 1)], sems.at[j % 8]
        ).start()
        pltpu.make_async_copy(
            iw_hbm.at[pl.ds(r_i, 1)], ibuf.at[pl.ds(j, 1)], sems.at[(j + 4) % 8]
        ).start()
    for j in range(CHT):
        pltpu.make_async_copy(
            uw_hbm.at[pl.ds(0, 1)], ubuf.at[pl.ds(j, 1)], sems.at[j % 8]
        ).wait()
        pltpu.make_async_copy(
            iw_hbm.at[pl.ds(0, 1)], ibuf.at[pl.ds(j, 1)], sems.at[(j + 4) % 8]
        ).wait()
    out_v[...] = jnp.sum(ubuf[...] * ibuf[...], axis=1)


def tc_kernel(user, item, user_emb_w, item_emb_w):
    grid = (B // CHT,)
    return pl.pallas_call(
        _tc_body,
        grid_spec=pltpu.PrefetchScalarGridSpec(
            num_scalar_prefetch=2,
            grid=grid,
            in_specs=[
                pl.BlockSpec(memory_space=pl.ANY),
                pl.BlockSpec(memory_space=pl.ANY),
            ],
            out_specs=pl.BlockSpec((CHT,), lambda k, u, i: (k,)),
            scratch_shapes=[
                pltpu.VMEM((CHT, D), jnp.float32),
                pltpu.VMEM((CHT, D), jnp.float32),
                pltpu.SemaphoreType.DMA((8,)),
            ],
        ),
        out_shape=jax.ShapeDtypeStruct((B,), jnp.float32),
    )(user.astype(jnp.int32), item.astype(jnp.int32), user_emb_w, item_emb_w)


kernel = tc_kernel


# speedup vs baseline: 1.1469x; 1.1469x over previous
import jax
import jax.numpy as jnp
from jax import lax
from jax.experimental import pallas as pl
from jax.experimental.pallas import tpu as pltpu
from jax.experimental.pallas import tpu_sc as plsc

NC = 2
NS = 16
L = 16
NW = NC * NS
B = 16384
D = 64
B_SC = 11264
B_TC = B - B_SC
BPW = B_SC // NW
NG = BPW // L
CHT = 256


def _mf_sc_body(user_hbm, item_hbm, uw_hbm, iw_hbm, out_hbm,
                uidx_v, iidx_v, du_v, di_v, out_v, sem):
    wid = lax.axis_index("s") * NC + lax.axis_index("c")
    base = wid * BPW
    pltpu.sync_copy(user_hbm.at[pl.ds(base, BPW)], uidx_v)
    pltpu.sync_copy(item_hbm.at[pl.ds(base, BPW)], iidx_v)
    lane = lax.iota(jnp.int32, L)

    NGH = NG // 2
    for h in range(2):
        hb = h * NGH * L

        def issue_body(g, carry):
            uvec = uidx_v[pl.ds(hb + g * L, L)]
            ivec = iidx_v[pl.ds(hb + g * L, L)]
            for j in range(L):
                r_u = uvec[j]
                r_i = ivec[j]
                pltpu.async_copy(
                    uw_hbm.at[r_u >> 3, pl.ds(r_u & 7, 1)],
                    du_v.at[g * L + j], sem)
                pltpu.async_copy(
                    iw_hbm.at[r_i >> 3, pl.ds(r_i & 7, 1)],
                    di_v.at[g * L + j], sem)
            return carry

        lax.fori_loop(0, NGH, issue_body, 0)

        def comp_body(g, carry):
            for j in range(2 * L):
                pltpu.make_async_copy(uw_hbm.at[0, pl.ds(0, 1)], du_v.at[0],
                                      sem).wait()
            bvec = g * L + lane
            zero = jnp.zeros((L,), jnp.int32)

            def t_body(t, acc):
                col = lax.bitwise_and(lane + t, D - 1)
                a = plsc.load_gather(du_v, [bvec, zero, col])
                b = plsc.load_gather(di_v, [bvec, zero, col])
                return acc + a * b

            acc = lax.fori_loop(0, D, t_body, jnp.zeros((L,), jnp.float32))
            out_v[pl.ds(hb + g * L, L)] = acc
            return carry

        lax.fori_loop(0, NGH, comp_body, 0)
    pltpu.sync_copy(out_v, out_hbm.at[pl.ds(base, BPW)])


def _mf_tc_body(uidx_s, iidx_s, uw_hbm, iw_hbm, out_v, ubuf, ibuf, sems):
    k = pl.program_id(0)
    for j in range(CHT):
        r_u = uidx_s[k * CHT + j]
        r_i = iidx_s[k * CHT + j]
        pltpu.make_async_copy(
            uw_hbm.at[pl.ds(r_u, 1)], ubuf.at[pl.ds(j, 1)], sems.at[j % 8]
        ).start()
        pltpu.make_async_copy(
            iw_hbm.at[pl.ds(r_i, 1)], ibuf.at[pl.ds(j, 1)], sems.at[(j + 4) % 8]
        ).start()
    for s in range(8):
        pltpu.make_async_copy(
            uw_hbm.at[pl.ds(0, CHT // 8)], ubuf.at[pl.ds(0, CHT // 8)],
            sems.at[s]
        ).wait()
        pltpu.make_async_copy(
            uw_hbm.at[pl.ds(0, CHT // 8)], ubuf.at[pl.ds(0, CHT // 8)],
            sems.at[s]
        ).wait()
    out_v[...] = jnp.sum(ubuf[...] * ibuf[...], axis=1)


def kernel(user, item, user_emb_w, item_emb_w):
    user = user.astype(jnp.int32)
    item = item.astype(jnp.int32)
    nq = user_emb_w.shape[0] // 8
    uw3 = user_emb_w.reshape(nq, 8, D)
    iw3 = item_emb_w.reshape(nq, 8, D)

    mesh = plsc.VectorSubcoreMesh(core_axis_name="c", subcore_axis_name="s")
    sc_f = pl.kernel(
        _mf_sc_body,
        out_type=jax.ShapeDtypeStruct((B_SC,), jnp.float32),
        mesh=mesh,
        scratch_types=[
            pltpu.VMEM((BPW,), jnp.int32),
            pltpu.VMEM((BPW,), jnp.int32),
            pltpu.VMEM((BPW // 2, 1, D), jnp.float32),
            pltpu.VMEM((BPW // 2, 1, D), jnp.float32),
            pltpu.VMEM((BPW,), jnp.float32),
            pltpu.SemaphoreType.DMA,
        ],
        compiler_params=pltpu.CompilerParams(needs_layout_passes=False),
    )
    out_sc = sc_f(user[:B_SC], item[:B_SC], uw3, iw3)

    out_tc = pl.pallas_call(
        _mf_tc_body,
        grid_spec=pltpu.PrefetchScalarGridSpec(
            num_scalar_prefetch=2,
            grid=(B_TC // CHT,),
            in_specs=[
                pl.BlockSpec(memory_space=pl.ANY),
                pl.BlockSpec(memory_space=pl.ANY),
            ],
            out_specs=pl.BlockSpec((CHT,), lambda k, u, i: (k,)),
            scratch_shapes=[
                pltpu.VMEM((CHT, D), jnp.float32),
                pltpu.VMEM((CHT, D), jnp.float32),
                pltpu.SemaphoreType.DMA((8,)),
            ],
        ),
        out_shape=jax.ShapeDtypeStruct((B_TC,), jnp.float32),
    )(user[B_SC:], item[B_SC:], user_emb_w, item_emb_w)

    return jnp.concatenate([out_sc, out_tc])


# speedup vs baseline: 1.8842x; 1.6429x over previous
import jax
import jax.numpy as jnp
from jax import lax
from jax.experimental import pallas as pl
from jax.experimental.pallas import tpu as pltpu
from jax.experimental.pallas import tpu_sc as plsc

NC = 2
NS = 16
L = 16
NW = NC * NS
B = 16384
D = 64
BPW = B // NW
NG = BPW // L


def _mf_body(user_hbm, item_hbm, uw_hbm, iw_hbm, out_hbm,
             uidx_v, iidx_v, du_v, di_v, out_v, sem):
    wid = lax.axis_index("s") * NC + lax.axis_index("c")
    base = wid * BPW
    pltpu.sync_copy(user_hbm.at[pl.ds(base, BPW)], uidx_v)
    pltpu.sync_copy(item_hbm.at[pl.ds(base, BPW)], iidx_v)
    lane = lax.iota(jnp.int32, L)

    NGH = NG // 2
    for h in range(2):
        hb = h * NGH * L

        def issue_body(g, carry):
            uvec = uidx_v[pl.ds(hb + g * L, L)]
            ivec = iidx_v[pl.ds(hb + g * L, L)]
            for j in range(L):
                r_u = uvec[j]
                r_i = ivec[j]
                pltpu.async_copy(
                    uw_hbm.at[r_u >> 3, pl.ds(r_u & 7, 1)],
                    du_v.at[g * L + j], sem)
                pltpu.async_copy(
                    iw_hbm.at[r_i >> 3, pl.ds(r_i & 7, 1)],
                    di_v.at[g * L + j], sem)
            return carry

        lax.fori_loop(0, NGH, issue_body, 0)

        def comp_body(g, carry):
            for j in range(2 * L):
                pltpu.make_async_copy(uw_hbm.at[0, pl.ds(0, 1)], du_v.at[0],
                                      sem).wait()
            bvec = g * L + lane
            zero = jnp.zeros((L,), jnp.int32)

            def t_body(t, acc):
                col = lax.bitwise_and(lane + t, D - 1)
                a = plsc.load_gather(du_v, [bvec, zero, col])
                b = plsc.load_gather(di_v, [bvec, zero, col])
                return acc + a * b

            acc = lax.fori_loop(0, D, t_body, jnp.zeros((L,), jnp.float32))
            out_v[pl.ds(hb + g * L, L)] = acc
            return carry

        lax.fori_loop(0, NGH, comp_body, 0)
    pltpu.sync_copy(out_v, out_hbm.at[pl.ds(base, BPW)])


def kernel(user, item, user_emb_w, item_emb_w):
    mesh = plsc.VectorSubcoreMesh(core_axis_name="c", subcore_axis_name="s")
    f = pl.kernel(
        _mf_body,
        out_type=jax.ShapeDtypeStruct((B,), jnp.float32),
        mesh=mesh,
        scratch_types=[
            pltpu.VMEM((BPW,), jnp.int32),
            pltpu.VMEM((BPW,), jnp.int32),
            pltpu.VMEM((BPW // 2, 1, D), jnp.float32),
            pltpu.VMEM((BPW // 2, 1, D), jnp.float32),
            pltpu.VMEM((BPW,), jnp.float32),
            pltpu.SemaphoreType.DMA,
        ],
        compiler_params=pltpu.CompilerParams(needs_layout_passes=False),
    )
    nq = user_emb_w.shape[0] // 8
    return f(user.astype(jnp.int32), item.astype(jnp.int32),
             user_emb_w.reshape(nq, 8, D), item_emb_w.reshape(nq, 8, D))
